# Initial kernel scaffold; baseline (speedup 1.0000x reference)
import jax
import jax.numpy as jnp
from jax.experimental import pallas as pl


def kernel(item_ids, language_table, id_table):
    raise NotImplementedError("write your pallas kernel here")


# speedup vs baseline: 4.0313x; 4.0313x over previous
import functools

import jax
import jax.numpy as jnp
from jax import lax
from jax.experimental import pallas as pl
from jax.experimental.pallas import tpu as pltpu
from jax.experimental.pallas import tpu_sc as plsc

HIDDEN = 128
ID_DIM = 64
LANG_OFF = HIDDEN - ID_DIM
LANES = 16
CHUNK = 128


def kernel(item_ids, language_table, id_table):
    B, H = item_ids.shape
    N = B * H
    ids_flat = item_ids.reshape(N)

    info = plsc.get_sparse_core_info()
    NC, NS = info.num_cores, info.num_subcores
    NW = NC * NS
    per_w = N // NW
    n_chunks = per_w // CHUNK

    @functools.partial(
        pl.kernel,
        mesh=plsc.VectorSubcoreMesh(core_axis_name="c", subcore_axis_name="s"),
        out_type=jax.ShapeDtypeStruct((N, HIDDEN), jnp.float32),
        scratch_types=[
            pltpu.VMEM((CHUNK,), jnp.int32),
            pltpu.VMEM((CHUNK, HIDDEN), jnp.float32),
            pltpu.VMEM((CHUNK, ID_DIM), jnp.float32),
            pltpu.SemaphoreType.DMA,
            pltpu.SemaphoreType.DMA,
        ],
        compiler_params=pltpu.CompilerParams(use_tc_tiling_on_sc=False),
    )
    def run(ids_hbm, lang_hbm, id_hbm, out_hbm, idx_v, buf, buf_id, sem1, sem2):
        wid = lax.axis_index("s") * NC + lax.axis_index("c")
        base = wid * per_w

        def chunk_body(c, carry):
            row0 = base + c * CHUNK
            pltpu.sync_copy(ids_hbm.at[pl.ds(row0, CHUNK)], idx_v)
            g1 = pltpu.async_copy(lang_hbm.at[idx_v], buf, sem1)
            g2 = pltpu.async_copy(id_hbm.at[idx_v], buf_id, sem2)
            g1.wait()
            g2.wait()

            def add_row(r, rcarry):
                for cc in range(ID_DIM // LANES):
                    plsc.addupdate(
                        buf.at[r, pl.ds(LANG_OFF + cc * LANES, LANES)],
                        buf_id[r, pl.ds(cc * LANES, LANES)],
                    )
                return rcarry

            lax.fori_loop(0, CHUNK, add_row, 0)
            pltpu.sync_copy(buf, out_hbm.at[pl.ds(row0, CHUNK)])
            return carry

        lax.fori_loop(0, n_chunks, chunk_body, 0)

    out = run(ids_flat, language_table, id_table)
    return out.reshape(B, H, HIDDEN)


# speedup vs baseline: 5.5691x; 1.3815x over previous
import functools

import jax
import jax.numpy as jnp
from jax import lax
from jax.experimental import pallas as pl
from jax.experimental.pallas import tpu as pltpu
from jax.experimental.pallas import tpu_sc as plsc

HIDDEN = 128
ID_DIM = 64
LANG_OFF = HIDDEN - ID_DIM
LANES = 16
CHUNK = 80
NBUF = 4
AHEAD = 3


def kernel(item_ids, language_table, id_table):
    B, H = item_ids.shape
    N = B * H
    ids_flat = item_ids.reshape(N)

    info = plsc.get_sparse_core_info()
    NC, NS = info.num_cores, info.num_subcores
    NW = NC * NS
    per_w = N // NW
    n_chunks = per_w // CHUNK

    @functools.partial(
        pl.kernel,
        mesh=plsc.VectorSubcoreMesh(core_axis_name="c", subcore_axis_name="s"),
        out_type=jax.ShapeDtypeStruct((N, HIDDEN), jnp.float32),
        scratch_types=(
            [pltpu.VMEM((per_w,), jnp.int32)]
            + [pltpu.VMEM((CHUNK, HIDDEN), jnp.float32) for _ in range(NBUF)]
            + [pltpu.VMEM((CHUNK, ID_DIM), jnp.float32) for _ in range(NBUF)]
            + [pltpu.SemaphoreType.DMA for _ in range(3 * NBUF)]
        ),
        compiler_params=pltpu.CompilerParams(use_tc_tiling_on_sc=False),
    )
    def run(ids_hbm, lang_hbm, id_hbm, out_hbm, idx_all, *rest):
        bufs = rest[0:NBUF]
        bufids = rest[NBUF:2 * NBUF]
        gl_sem = rest[2 * NBUF:3 * NBUF]
        gi_sem = rest[3 * NBUF:4 * NBUF]
        w_sem = rest[4 * NBUF:5 * NBUF]

        wid = lax.axis_index("s") * NC + lax.axis_index("c")
        base = wid * per_w
        pltpu.sync_copy(ids_hbm.at[pl.ds(base, per_w)], idx_all)

        def idx_slice(k):
            return idx_all.at[pl.ds(k * CHUNK, CHUNK)]

        def fire_gather(k, b):
            pltpu.async_copy(lang_hbm.at[idx_slice(k)], bufs[b], gl_sem[b])
            pltpu.async_copy(id_hbm.at[idx_slice(k)], bufids[b], gi_sem[b])

        def wait_gather(b):
            pltpu.make_async_copy(lang_hbm.at[idx_slice(0)], bufs[b], gl_sem[b]).wait()
            pltpu.make_async_copy(id_hbm.at[idx_slice(0)], bufids[b], gi_sem[b]).wait()

        def fuse_and_write(k, b):
            def add_rows(r4, carry):
                for u in range(4):
                    r = r4 * 4 + u
                    for cc in range(ID_DIM // LANES):
                        plsc.addupdate(
                            bufs[b].at[r, pl.ds(LANG_OFF + cc * LANES, LANES)],
                            bufids[b][r, pl.ds(cc * LANES, LANES)],
                        )
                return carry

            lax.fori_loop(0, CHUNK // 4, add_rows, 0)
            pltpu.async_copy(
                bufs[b], out_hbm.at[pl.ds(base + k * CHUNK, CHUNK)], w_sem[b]
            )

        def wait_write(k, b):
            pltpu.make_async_copy(
                bufs[b], out_hbm.at[pl.ds(base + k * CHUNK, CHUNK)], w_sem[b]
            ).wait()

        for b in range(AHEAD):
            fire_gather(b, b)

        for b in range(NBUF):
            wait_gather(b)
            fuse_and_write(b, b)
            kn = b + AHEAD
            if kn < NBUF:
                fire_gather(kn, kn % NBUF)
            else:
                bn = kn % NBUF
                wait_write(kn - NBUF, bn)
                fire_gather(kn, bn)

        def group_body(g, carry):
            for b in range(NBUF):
                k = g * NBUF + b
                wait_gather(b)
                fuse_and_write(k, b)
                kn = k + AHEAD
                bn = (b + AHEAD) % NBUF

                @pl.when(kn < n_chunks)
                def _():
                    wait_write(kn - NBUF, bn)
                    fire_gather(kn, bn)

            return carry

        lax.fori_loop(1, n_chunks // NBUF, group_body, 0)

        for b in range(NBUF):
            wait_write(n_chunks - NBUF + b, (n_chunks - NBUF + b) % NBUF)

    out = run(ids_flat, language_table, id_table)
    return out.reshape(B, H, HIDDEN)
